# baseline (speedup 1.0000x reference)
import functools

import jax
import jax.numpy as jnp
from jax import lax
from jax.experimental import pallas as pl
from jax.experimental.pallas import tpu as pltpu
from jax.experimental.pallas import tpu_sc as plsc

EMB_D = 32
NC, NS = 2, 16
NW = NC * NS
BW = 128
SG = 8
NBUF = 2
KD = EMB_D // 8


def _build(BATCH, SEQ):
    assert BATCH == NW * BW and SEQ % SG == 0
    ngroups = SEQ // SG
    assert ngroups >= 2 * NBUF
    npipe = (ngroups // NBUF) * NBUF

    mesh = plsc.VectorSubcoreMesh(core_axis_name="c", subcore_axis_name="s")

    @functools.partial(
        pl.kernel,
        mesh=mesh,
        compiler_params=pltpu.CompilerParams(
            use_tc_tiling_on_sc=False, needs_layout_passes=False),
        out_type=jax.ShapeDtypeStruct((SEQ, KD, NW * 8, BW), jnp.float32),
        scratch_types=[
            pltpu.VMEM((NBUF, SG, BW), jnp.int32),
            pltpu.VMEM((NBUF, SG * BW, EMB_D), jnp.float32),
            pltpu.VMEM((SG, KD, 8, BW), jnp.float32),
            pltpu.SemaphoreType.DMA,
            pltpu.SemaphoreType.DMA,
        ],
    )
    def emb(xT_hbm, w_hbm, out_hbm, idxr, rows_v, trans_v, gsem0, gsem1):
        gsem = (gsem0, gsem1)
        wid = lax.axis_index("s") * NC + lax.axis_index("c")
        b0 = wid * BW
        bvec = lax.iota(jnp.int32, 16)

        def load_idx(b, g):
            pltpu.sync_copy(
                xT_hbm.at[pl.ds(g * SG, SG), pl.ds(b0, BW)], idxr.at[b])

        def issue_gathers(b):
            for sp in range(SG):
                pltpu.async_copy(
                    w_hbm.at[idxr.at[b].at[sp]],
                    rows_v.at[b].at[pl.ds(sp * BW, BW)],
                    gsem[b])

        def drain_gathers(b):
            pltpu.make_async_copy(
                w_hbm.at[pl.ds(0, SG * BW)], rows_v.at[b], gsem[b]).wait()

        def transpose_group(b):
            rf = rows_v.at[b]

            @plsc.parallel_loop(0, SG * KD * 8, 1, unroll=8)
            def tbody(u):
                t = u // 8
                bg = u - t * 8
                sp = t // KD
                k = t - sp * KD
                rowb = bvec + (sp * BW + bg * 16)
                for dp in range(8):
                    col = jnp.full((16,), 0, jnp.int32) + (k * 8 + dp)
                    v = plsc.load_gather(rf, [rowb, col])
                    trans_v[sp, k, dp, pl.ds(bg * 16, 16)] = v

        def write_group(g):
            pltpu.sync_copy(
                trans_v,
                out_hbm.at[pl.ds(g * SG, SG), pl.ds(0, KD),
                           pl.ds(wid * 8, 8)])

        for b in range(NBUF):
            load_idx(b, b)
            issue_gathers(b)

        def body(i, carry):
            for b in range(NBUF):
                g = NBUF * i + b
                drain_gathers(b)
                transpose_group(b)
                write_group(g)
                load_idx(b, g + NBUF)
                issue_gathers(b)
            return carry

        lax.fori_loop(0, npipe // NBUF - 1, body, 0)

        for b in range(NBUF):
            g = npipe - NBUF + b
            drain_gathers(b)
            transpose_group(b)
            write_group(g)

        for g in range(npipe, ngroups):
            load_idx(0, g)
            issue_gathers(0)
            drain_gathers(0)
            transpose_group(0)
            write_group(g)

    return emb


def kernel(x, weight):
    b0, b1 = x.shape
    xT = jnp.swapaxes(x.astype(jnp.int32), 0, 1)
    r = _build(b0, b1)(xT, weight)
    r = r.reshape(b1, KD, NW, 8, BW)
    return r.transpose(2, 4, 0, 1, 3).reshape(b0, b1, EMB_D)


# speedup vs baseline: 1.0272x; 1.0272x over previous
import functools

import jax
import jax.numpy as jnp
from jax import lax
from jax.experimental import pallas as pl
from jax.experimental.pallas import tpu as pltpu
from jax.experimental.pallas import tpu_sc as plsc

EMB_D = 32
NC, NS = 2, 16
NW = NC * NS
BW = 128
SG = 8
NBUF = 2
KD = EMB_D // 8


def _build(BATCH, SEQ):
    assert BATCH == NW * BW and SEQ % SG == 0
    ngroups = SEQ // SG
    assert ngroups >= 2 * NBUF
    npipe = (ngroups // NBUF) * NBUF

    mesh = plsc.VectorSubcoreMesh(core_axis_name="c", subcore_axis_name="s")

    @functools.partial(
        pl.kernel,
        mesh=mesh,
        compiler_params=pltpu.CompilerParams(
            use_tc_tiling_on_sc=False, needs_layout_passes=False),
        out_type=jax.ShapeDtypeStruct((SEQ, KD, NW * 8, BW), jnp.float32),
        scratch_types=[
            pltpu.VMEM((NBUF, SG, BW), jnp.int32),
            pltpu.VMEM((NBUF, SG * BW, EMB_D), jnp.float32),
            pltpu.VMEM((SG, KD, 8, BW), jnp.float32),
            pltpu.SemaphoreType.DMA,
            pltpu.SemaphoreType.DMA,
        ],
    )
    def emb(xT_hbm, w_hbm, out_hbm, idxr, rows_v, trans_v, gsem0, gsem1):
        gsem = (gsem0, gsem1)
        wid = lax.axis_index("s") * NC + lax.axis_index("c")
        b0 = wid * BW
        bvec = lax.iota(jnp.int32, 16)

        def load_idx(b, g):
            pltpu.sync_copy(
                xT_hbm.at[pl.ds(g * SG, SG), pl.ds(b0, BW)], idxr.at[b])

        def issue_gathers(b):
            for sp in range(SG):
                pltpu.async_copy(
                    w_hbm.at[idxr.at[b].at[sp]],
                    rows_v.at[b].at[pl.ds(sp * BW, BW)],
                    gsem[b])

        def drain_gathers(b):
            pltpu.make_async_copy(
                w_hbm.at[pl.ds(0, SG * BW)], rows_v.at[b], gsem[b]).wait()

        def transpose_group(b):
            rf = rows_v.at[b]

            @plsc.parallel_loop(0, SG * KD, 1, unroll=2)
            def tbody(t):
                sp = t // KD
                k = t - sp * KD
                base = sp * BW
                for bg in range(BW // 16):
                    rowb = bvec + (base + bg * 16)
                    for dp in range(8):
                        col = jnp.full((16,), 0, jnp.int32) + (k * 8 + dp)
                        v = plsc.load_gather(rf, [rowb, col])
                        trans_v[sp, k, dp, pl.ds(bg * 16, 16)] = v

        def write_group(g):
            pltpu.sync_copy(
                trans_v,
                out_hbm.at[pl.ds(g * SG, SG), pl.ds(0, KD),
                           pl.ds(wid * 8, 8)])

        for b in range(NBUF):
            load_idx(b, b)
            issue_gathers(b)

        def body(i, carry):
            for b in range(NBUF):
                g = NBUF * i + b
                drain_gathers(b)
                transpose_group(b)
                write_group(g)
                load_idx(b, g + NBUF)
                issue_gathers(b)
            return carry

        lax.fori_loop(0, npipe // NBUF - 1, body, 0)

        for b in range(NBUF):
            g = npipe - NBUF + b
            drain_gathers(b)
            transpose_group(b)
            write_group(g)

        for g in range(npipe, ngroups):
            load_idx(0, g)
            issue_gathers(0)
            drain_gathers(0)
            transpose_group(0)
            write_group(g)

    return emb


def kernel(x, weight):
    b0, b1 = x.shape
    xT = jnp.swapaxes(x.astype(jnp.int32), 0, 1)
    r = _build(b0, b1)(xT, weight)
    r = r.reshape(b1, KD, NW, 8, BW)
    return r.transpose(2, 4, 0, 1, 3).reshape(b0, b1, EMB_D)
